# baseline (speedup 1.0000x reference)
import functools

import jax
import jax.numpy as jnp
from jax import lax
from jax.experimental import pallas as pl
from jax.experimental.pallas import tpu as pltpu
from jax.experimental.pallas import tpu_sc as plsc

_NC = 2
_NS = 16
_NW = _NC * _NS

_NB = 3


@functools.lru_cache(maxsize=None)
def _build(bsz, seqlen, d, dpad):
    seq_per_w = bsz // _NW
    per_w = seq_per_w * seqlen
    c0 = 128
    c1 = seqlen - c0
    ngroup = seq_per_w // _NB
    mesh = plsc.VectorSubcoreMesh(core_axis_name="c", subcore_axis_name="s")

    @functools.partial(
        pl.kernel,
        mesh=mesh,
        out_type=jax.ShapeDtypeStruct((bsz, seqlen, dpad), jnp.float32),
        scratch_types=[
            pltpu.VMEM((per_w,), jnp.int32),
            pltpu.VMEM((_NB, seqlen, dpad), jnp.float32),
            pltpu.SemaphoreType.DMA((_NB,)),
            pltpu.SemaphoreType.DMA((_NB,)),
        ],
        compiler_params=pltpu.CompilerParams(use_tc_tiling_on_sc=True),
    )
    def emb(table_hbm, idx_hbm, out_hbm, idx_v, rows_v, gsem, wsem):
        wid = lax.axis_index("s") * _NC + lax.axis_index("c")
        sbase = wid * seq_per_w
        pltpu.sync_copy(idx_hbm.at[pl.ds(sbase * seqlen, per_w)], idx_v)

        def start_gathers(s, b):
            off = s * seqlen
            pltpu.async_copy(
                table_hbm.at[idx_v.at[pl.ds(off, c0)]],
                rows_v.at[b, pl.ds(0, c0)],
                gsem.at[b],
            )
            pltpu.async_copy(
                table_hbm.at[idx_v.at[pl.ds(off + c0, c1)]],
                rows_v.at[b, pl.ds(c0, c1)],
                gsem.at[b],
            )

        def wait_gathers(b):
            pltpu.make_async_copy(
                table_hbm.at[pl.ds(0, seqlen)], rows_v.at[b], gsem.at[b]
            ).wait()

        def start_write(s, b):
            pltpu.async_copy(
                rows_v.at[b], out_hbm.at[sbase + s], wsem.at[b]
            )

        def wait_write(b):
            pltpu.make_async_copy(
                rows_v.at[b], out_hbm.at[sbase], wsem.at[b]
            ).wait()

        for b in range(_NB):
            start_gathers(b, b)

        def outer(g, carry):
            s0 = g * _NB
            for b in range(_NB):
                wait_gathers(b)
                start_write(s0 + b, b)
            for b in range(_NB):
                wait_write(b)
                start_gathers(s0 + _NB + b, b)
            return carry

        lax.fori_loop(0, ngroup - 1, outer, 0)

        s0 = (ngroup - 1) * _NB
        for b in range(_NB):
            wait_gathers(b)
            start_write(s0 + b, b)
        for b in range(_NB):
            wait_write(b)

    return emb


def kernel(table, seq):
    b, l = seq.shape
    v, d = table.shape
    dpad = 128
    padded = jnp.pad(table, ((0, 0), (0, dpad - d)))
    idx = seq.reshape(-1).astype(jnp.int32)
    out = _build(b, l, d, dpad)(padded, idx)
    return out[:, :, :d]
